# baseline (speedup 1.0000x reference)
import jax
import jax.numpy as jnp
from jax.experimental import pallas as pl

_B = 1024
_HP_W = 128 * 1 * 3
_QT_W = 128 * 52 * 4
_G = 16


def _copy_body(hp_ref, qt_ref, hp_out, qt_out):
    hp_out[...] = hp_ref[...]
    qt_out[...] = qt_ref[...]


def kernel(hip_pos, quat):
    hp = hip_pos.reshape(_B, _HP_W)
    qt = quat.reshape(_B, _QT_W)
    hp_o, qt_o = pl.pallas_call(
        _copy_body,
        grid=(_B // _G,),
        in_specs=[
            pl.BlockSpec((_G, _HP_W), lambda i: (i, 0)),
            pl.BlockSpec((_G, _QT_W), lambda i: (i, 0)),
        ],
        out_specs=[
            pl.BlockSpec((_G, _HP_W), lambda i: (i, 0)),
            pl.BlockSpec((_G, _QT_W), lambda i: (i, 0)),
        ],
        out_shape=[
            jax.ShapeDtypeStruct((_B, _HP_W), hip_pos.dtype),
            jax.ShapeDtypeStruct((_B, _QT_W), quat.dtype),
        ],
    )(hp, qt)
    return hp_o.reshape(hip_pos.shape), qt_o.reshape(quat.shape)
